# baseline (speedup 1.0000x reference)
import functools

import jax
import jax.numpy as jnp
from jax import lax
from jax.experimental import pallas as pl
from jax.experimental.pallas import tpu as pltpu
from jax.experimental.pallas import tpu_sc as plsc

_CHUNK = 128
_W = 512


@functools.lru_cache(maxsize=None)
def _build(n_user, n_item, batch, embed):
    info = plsc.get_sparse_core_info()
    num_cores = info.num_cores
    num_workers = info.num_cores * info.num_subcores
    assert batch % (num_workers * _CHUNK) == 0
    b_per_w = batch // num_workers
    n_chunks = b_per_w // _CHUNK
    total = 2 * batch

    def tr_body(x_ref, o_ref):
        x = x_ref[...]
        ir = lax.broadcasted_iota(jnp.int32, (embed, embed), 0)
        ic = lax.broadcasted_iota(jnp.int32, (embed, embed), 1)
        ident = (ir == ic).astype(jnp.float32)
        dn = (((0,), (0,)), ((), ()))
        ts = [
            lax.dot_general(x[:, embed * g:embed * (g + 1)], ident, dn,
                            preferred_element_type=jnp.float32)
            for g in range(_W // embed)
        ]
        t = jnp.concatenate(ts, axis=0)
        o_ref[...] = jnp.concatenate([t[:_W // 2], t[_W // 2:]], axis=-1)

    def make_tr(n):
        grid = (n + _W - 1) // _W
        return pl.pallas_call(
            tr_body,
            grid=(grid,),
            in_specs=[pl.BlockSpec((embed, _W), lambda j: (0, j))],
            out_specs=pl.BlockSpec((_W // 2, 2 * embed), lambda j: (j, 0)),
            out_shape=jax.ShapeDtypeStruct((grid * _W // 2, 2 * embed),
                                           jnp.float32),
        )

    tr_u = make_tr(n_user)
    tr_i = make_tr(n_item)

    mesh = plsc.VectorSubcoreMesh(core_axis_name="c", subcore_axis_name="s")

    @functools.partial(
        pl.kernel,
        mesh=mesh,
        out_type=jax.ShapeDtypeStruct((total, 2 * embed), jnp.float32),
        scratch_types=[
            pltpu.VMEM((n_chunks, _CHUNK), jnp.int32),
            pltpu.VMEM((n_chunks, _CHUNK), jnp.int32),
            pltpu.VMEM((b_per_w, 2 * embed), jnp.float32),
            pltpu.SemaphoreType.DMA,
            pltpu.SemaphoreType.DMA,
        ],
    )
    def gather_sc(user_hbm, item_hbm, pidx_u_hbm, pidx_i_hbm, out_hbm,
                  idx_u_v, idx_i_v, buf, gsem, wsem):
        wid = lax.axis_index("s") * num_cores + lax.axis_index("c")
        base = wid * b_per_w

        pltpu.sync_copy(pidx_u_hbm.at[wid], idx_u_v)
        pltpu.sync_copy(pidx_i_hbm.at[wid], idx_i_v)

        copies = [
            pltpu.async_copy(
                user_hbm.at[idx_u_v.at[c]],
                buf.at[pl.ds(c * _CHUNK, _CHUNK)],
                gsem,
            )
            for c in range(n_chunks)
        ]
        for cp in copies:
            cp.wait()
        w = pltpu.async_copy(buf, out_hbm.at[pl.ds(base, b_per_w)], wsem)
        w.wait()

        copies = [
            pltpu.async_copy(
                item_hbm.at[idx_i_v.at[c]],
                buf.at[pl.ds(c * _CHUNK, _CHUNK)],
                gsem,
            )
            for c in range(n_chunks)
        ]
        for cp in copies:
            cp.wait()
        w = pltpu.async_copy(
            buf, out_hbm.at[pl.ds(batch + base, b_per_w)], wsem)
        w.wait()

    blk = 2048
    n_blk = total // blk

    def select_tc(rows_ref, bits_ref, o_ref):
        r = rows_ref[...]
        b = bits_ref[...] > 0
        o_ref[...] = jnp.where(b, r[:, embed:], r[:, :embed])

    select = pl.pallas_call(
        select_tc,
        grid=(n_blk,),
        in_specs=[
            pl.BlockSpec((blk, 2 * embed), lambda i: (i, 0)),
            pl.BlockSpec((blk, 1), lambda i: (i, 0)),
        ],
        out_specs=pl.BlockSpec((blk, embed), lambda i: (i, 0)),
        out_shape=jax.ShapeDtypeStruct((total, embed), jnp.float32),
    )

    def call(embed_user, embed_item, idx_user, idx_item):
        scr_u = tr_u(embed_user.T)
        scr_i = tr_i(embed_item.T)
        idx_u = idx_user.astype(jnp.int32)
        idx_i = idx_item.astype(jnp.int32)
        half = _W // 2

        def phys(idx):
            return (idx // _W) * half + (idx % half)

        pidx_u = phys(idx_u).reshape(num_workers, n_chunks, _CHUNK)
        pidx_i = phys(idx_i).reshape(num_workers, n_chunks, _CHUNK)
        bit_u = (idx_u // half) & 1
        bit_i = (idx_i // half) & 1
        bits = jnp.concatenate([bit_u, bit_i]).reshape(total, 1)
        rows = gather_sc(scr_u, scr_i, pidx_u, pidx_i)
        return select(rows, bits)

    return call


def kernel(embed_user, embed_item, idx_user, idx_item):
    n_user, embed = embed_user.shape
    n_item = embed_item.shape[0]
    batch = idx_user.shape[0]
    return _build(n_user, n_item, batch, embed)(
        embed_user, embed_item, idx_user, idx_item)


# speedup vs baseline: 2.2667x; 2.2667x over previous
import functools

import jax
import jax.numpy as jnp
from jax import lax
from jax.experimental import pallas as pl
from jax.experimental.pallas import tpu as pltpu
from jax.experimental.pallas import tpu_sc as plsc

_CHUNK = 128


@functools.lru_cache(maxsize=None)
def _build(n_user, n_item, batch, embed):
    info = plsc.get_sparse_core_info()
    num_cores = info.num_cores
    num_workers = info.num_cores * info.num_subcores
    assert batch % (num_workers * _CHUNK) == 0
    b_per_w = batch // num_workers
    n_chunks = b_per_w // _CHUNK

    mesh = plsc.VectorSubcoreMesh(core_axis_name="c", subcore_axis_name="s")

    @functools.partial(
        pl.kernel,
        mesh=mesh,
        out_type=jax.ShapeDtypeStruct((2 * batch, embed), jnp.float32),
        compiler_params=pltpu.CompilerParams(use_tc_tiling_on_sc=False),
        scratch_types=[
            pltpu.VMEM((n_chunks, _CHUNK), jnp.int32),
            pltpu.VMEM((n_chunks, _CHUNK), jnp.int32),
            pltpu.VMEM((b_per_w, embed), jnp.float32),
            pltpu.VMEM((b_per_w, embed), jnp.float32),
            pltpu.SemaphoreType.DMA,
            pltpu.SemaphoreType.DMA,
            pltpu.SemaphoreType.DMA,
        ],
    )
    def run(user_hbm, item_hbm, idx_u_hbm, idx_i_hbm, out_hbm,
            idx_u_v, idx_i_v, rows_u, rows_i, usem, isem, wsem):
        wid = lax.axis_index("s") * num_cores + lax.axis_index("c")
        base = wid * b_per_w

        pltpu.sync_copy(idx_u_hbm.at[wid], idx_u_v)
        pltpu.sync_copy(idx_i_hbm.at[wid], idx_i_v)

        u_copies = [
            pltpu.async_copy(
                user_hbm.at[idx_u_v.at[c]],
                rows_u.at[pl.ds(c * _CHUNK, _CHUNK)],
                usem,
            )
            for c in range(n_chunks)
        ]
        i_copies = [
            pltpu.async_copy(
                item_hbm.at[idx_i_v.at[c]],
                rows_i.at[pl.ds(c * _CHUNK, _CHUNK)],
                isem,
            )
            for c in range(n_chunks)
        ]

        for cp in u_copies:
            cp.wait()
        w_u = pltpu.async_copy(rows_u, out_hbm.at[pl.ds(base, b_per_w)], wsem)
        for cp in i_copies:
            cp.wait()
        w_i = pltpu.async_copy(
            rows_i, out_hbm.at[pl.ds(batch + base, b_per_w)], wsem)
        w_u.wait()
        w_i.wait()

    def call(embed_user, embed_item, idx_user, idx_item):
        idx_u = idx_user.astype(jnp.int32).reshape(num_workers, n_chunks, _CHUNK)
        idx_i = idx_item.astype(jnp.int32).reshape(num_workers, n_chunks, _CHUNK)
        return run(embed_user, embed_item, idx_u, idx_i)

    return call


def kernel(embed_user, embed_item, idx_user, idx_item):
    n_user, embed = embed_user.shape
    n_item = embed_item.shape[0]
    batch = idx_user.shape[0]
    return _build(n_user, n_item, batch, embed)(
        embed_user, embed_item, idx_user, idx_item)
